# baseline (speedup 1.0000x reference)
import functools

import jax
import jax.numpy as jnp
from jax import lax
from jax.experimental import pallas as pl
from jax.experimental.pallas import tpu as pltpu
from jax.experimental.pallas import tpu_sc as plsc

VOCAB = 1000
D = 512
B_TOTAL = 4096 * 20


def _project_body(emb_ref, w_ref, b_ref, out_ref):
    p = lax.dot_general(
        emb_ref[...], w_ref[...],
        dimension_numbers=(((1,), (1,)), ((), ())),
        preferred_element_type=jnp.float32,
    )
    out_ref[...] = p + b_ref[...]


def _project_table(emb_table, W, b):
    return pl.pallas_call(
        _project_body,
        out_shape=jax.ShapeDtypeStruct((VOCAB, D), jnp.float32),
    )(emb_table, W, b.reshape(1, D))


_NW = 32
_NROW = 4096
_L = 20
_LPAD = 24
_B_PAD = _NROW * _LPAD
_B_PER_W = _B_PAD // _NW
_CHUNK = 96
_NCHUNK = _B_PER_W // _CHUNK


def _make_gather():
    mesh = plsc.VectorSubcoreMesh(core_axis_name="c", subcore_axis_name="s")

    @functools.partial(
        pl.kernel,
        mesh=mesh,
        out_type=jax.ShapeDtypeStruct((_B_PAD, D), jnp.float32),
        scratch_types=[
            pltpu.VMEM((_B_PER_W,), jnp.int32),
            pltpu.VMEM((2, _CHUNK, D), jnp.float32),
            pltpu.SemaphoreType.DMA,
            pltpu.SemaphoreType.DMA,
        ],
    )
    def gather_kernel(table_hbm, idx_hbm, out_hbm, idx_v, rows_v, gsem, wsem):
        wid = lax.axis_index("s") * 2 + lax.axis_index("c")
        base = wid * _B_PER_W
        pltpu.sync_copy(idx_hbm.at[pl.ds(base, _B_PER_W)], idx_v)

        def gcopy(g, slot):
            return pltpu.make_async_copy(
                table_hbm.at[idx_v.at[pl.ds(g * _CHUNK, _CHUNK)]],
                rows_v.at[slot], gsem)

        def wcopy(g, slot):
            return pltpu.make_async_copy(
                rows_v.at[slot],
                out_hbm.at[pl.ds(base + g * _CHUNK, _CHUNK)], wsem)

        gcopy(0, 0).start()
        gcopy(0, 0).wait()
        wcopy(0, 0).start()
        gcopy(1, 1).start()

        def body(g, _):
            slot = g % 2
            gcopy(g, slot).wait()
            wcopy(g, slot).start()
            wcopy(g - 1, 1 - slot).wait()
            gcopy(g + 1, 1 - slot).start()
            return 0

        lax.fori_loop(1, _NCHUNK - 1, body, 0)

        g_last = _NCHUNK - 1
        s_last = g_last % 2
        gcopy(g_last, s_last).wait()
        wcopy(g_last, s_last).start()
        wcopy(g_last - 1, 1 - s_last).wait()
        wcopy(g_last, s_last).wait()

    return gather_kernel


_BR = 64
_RL_GRID = _NROW // _BR


def _relayout_body(in_ref, out_ref):
    x = in_ref[...].reshape(_BR, _LPAD, D)
    out_ref[...] = x[:, :_L, :]


def _relayout(flat):
    return pl.pallas_call(
        _relayout_body,
        grid=(_RL_GRID,),
        in_specs=[pl.BlockSpec((_BR * _LPAD, D), lambda i: (i, 0))],
        out_specs=pl.BlockSpec((_BR, _L, D), lambda i: (i, 0, 0)),
        out_shape=jax.ShapeDtypeStruct((_NROW, _L, D), jnp.float32),
    )(flat)


def kernel(notation_ids, emb_table, W, b):
    P = _project_table(emb_table, W, b)
    ids = notation_ids.astype(jnp.int32)
    ids_pad = jnp.concatenate(
        [ids, jnp.zeros((_NROW, _LPAD - _L), jnp.int32)], axis=1)
    out_pad = _make_gather()(P, ids_pad.reshape(-1))
    return _relayout(out_pad)


# speedup vs baseline: 1.0004x; 1.0004x over previous
import functools

import jax
import jax.numpy as jnp
from jax import lax
from jax.experimental import pallas as pl
from jax.experimental.pallas import tpu as pltpu
from jax.experimental.pallas import tpu_sc as plsc

VOCAB = 1000
D = 512
B_TOTAL = 4096 * 20


def _project_body(emb_ref, w_ref, b_ref, out_ref):
    p = lax.dot_general(
        emb_ref[...], w_ref[...],
        dimension_numbers=(((1,), (1,)), ((), ())),
        preferred_element_type=jnp.float32,
    )
    out_ref[...] = p + b_ref[...]


def _project_table(emb_table, W, b):
    return pl.pallas_call(
        _project_body,
        out_shape=jax.ShapeDtypeStruct((VOCAB, D), jnp.float32),
    )(emb_table, W, b.reshape(1, D))


_NW = 32
_NROW = 4096
_L = 20
_LPAD = 24
_B_PAD = _NROW * _LPAD
_B_PER_W = _B_PAD // _NW
_CHUNK = 64
_NCHUNK = _B_PER_W // _CHUNK


def _make_gather():
    mesh = plsc.VectorSubcoreMesh(core_axis_name="c", subcore_axis_name="s")

    @functools.partial(
        pl.kernel,
        mesh=mesh,
        out_type=jax.ShapeDtypeStruct((_B_PAD, D), jnp.float32),
        scratch_types=[
            pltpu.VMEM((_B_PER_W,), jnp.int32),
            pltpu.VMEM((2, _CHUNK, D), jnp.float32),
            pltpu.SemaphoreType.DMA,
            pltpu.SemaphoreType.DMA,
        ],
    )
    def gather_kernel(table_hbm, idx_hbm, out_hbm, idx_v, rows_v, gsem, wsem):
        wid = lax.axis_index("s") * 2 + lax.axis_index("c")
        base = wid * _B_PER_W
        pltpu.sync_copy(idx_hbm.at[pl.ds(base, _B_PER_W)], idx_v)

        def gcopy(g, slot):
            return pltpu.make_async_copy(
                table_hbm.at[idx_v.at[pl.ds(g * _CHUNK, _CHUNK)]],
                rows_v.at[slot], gsem)

        def wcopy(g, slot):
            return pltpu.make_async_copy(
                rows_v.at[slot],
                out_hbm.at[pl.ds(base + g * _CHUNK, _CHUNK)], wsem)

        gcopy(0, 0).start()
        gcopy(0, 0).wait()
        wcopy(0, 0).start()
        gcopy(1, 1).start()

        def body(g, _):
            slot = g % 2
            gcopy(g, slot).wait()
            wcopy(g, slot).start()
            wcopy(g - 1, 1 - slot).wait()
            gcopy(g + 1, 1 - slot).start()
            return 0

        lax.fori_loop(1, _NCHUNK - 1, body, 0)

        g_last = _NCHUNK - 1
        s_last = g_last % 2
        gcopy(g_last, s_last).wait()
        wcopy(g_last, s_last).start()
        wcopy(g_last - 1, 1 - s_last).wait()
        wcopy(g_last, s_last).wait()

    return gather_kernel


_BR = 64
_RL_GRID = _NROW // _BR


def _relayout_body(in_ref, out_ref):
    x = in_ref[...].reshape(_BR, _LPAD, D)
    out_ref[...] = x[:, :_L, :]


def _relayout(flat):
    return pl.pallas_call(
        _relayout_body,
        grid=(_RL_GRID,),
        in_specs=[pl.BlockSpec((_BR * _LPAD, D), lambda i: (i, 0))],
        out_specs=pl.BlockSpec((_BR, _L, D), lambda i: (i, 0, 0)),
        out_shape=jax.ShapeDtypeStruct((_NROW, _L, D), jnp.float32),
    )(flat)


def kernel(notation_ids, emb_table, W, b):
    P = _project_table(emb_table, W, b)
    ids = notation_ids.astype(jnp.int32)
    ids_pad = jnp.concatenate(
        [ids, jnp.zeros((_NROW, _LPAD - _L), jnp.int32)], axis=1)
    out_pad = _make_gather()(P, ids_pad.reshape(-1))
    return _relayout(out_pad)


# speedup vs baseline: 8.1670x; 8.1637x over previous
import functools

import jax
import jax.numpy as jnp
from jax import lax
from jax.experimental import pallas as pl
from jax.experimental.pallas import tpu as pltpu
from jax.experimental.pallas import tpu_sc as plsc

VOCAB = 1000
D = 512
_NROW = 4096
_L = 20
B_TOTAL = _NROW * _L


def _project_body(emb_ref, w_ref, b_ref, out_ref):
    p = lax.dot_general(
        emb_ref[...], w_ref[...],
        dimension_numbers=(((1,), (1,)), ((), ())),
        preferred_element_type=jnp.float32,
    )
    out_ref[...] = p + b_ref[...]


def _project_table(emb_table, W, b):
    return pl.pallas_call(
        _project_body,
        out_shape=jax.ShapeDtypeStruct((VOCAB, D), jnp.float32),
    )(emb_table, W, b.reshape(1, D))


_NW = 32
_B_PER_W = B_TOTAL // _NW
_CHUNK = 64
_NCHUNK = _B_PER_W // _CHUNK
_CPP = _NROW // _CHUNK


def _make_gather():
    mesh = plsc.VectorSubcoreMesh(core_axis_name="c", subcore_axis_name="s")

    @functools.partial(
        pl.kernel,
        mesh=mesh,
        out_type=jax.ShapeDtypeStruct((_L, _NROW, D), jnp.float32),
        scratch_types=[
            pltpu.VMEM((_B_PER_W,), jnp.int32),
            pltpu.VMEM((2, _CHUNK, D), jnp.float32),
            pltpu.SemaphoreType.DMA,
            pltpu.SemaphoreType.DMA,
        ],
    )
    def gather_kernel(table_hbm, idx_hbm, out_hbm, idx_v, rows_v, gsem, wsem):
        wid = lax.axis_index("s") * 2 + lax.axis_index("c")
        base = wid * _B_PER_W
        kappa0 = wid * _NCHUNK
        pltpu.sync_copy(idx_hbm.at[pl.ds(base, _B_PER_W)], idx_v)

        def gcopy(g, slot):
            return pltpu.make_async_copy(
                table_hbm.at[idx_v.at[pl.ds(g * _CHUNK, _CHUNK)]],
                rows_v.at[slot], gsem)

        def wcopy(g, slot):
            kappa = kappa0 + g
            j = kappa // _CPP
            i0 = (kappa % _CPP) * _CHUNK
            return pltpu.make_async_copy(
                rows_v.at[slot],
                out_hbm.at[j, pl.ds(i0, _CHUNK)], wsem)

        gcopy(0, 0).start()
        gcopy(0, 0).wait()
        wcopy(0, 0).start()
        gcopy(1, 1).start()

        def body(g, _):
            slot = g % 2
            gcopy(g, slot).wait()
            wcopy(g, slot).start()
            wcopy(g - 1, 1 - slot).wait()
            gcopy(g + 1, 1 - slot).start()
            return 0

        lax.fori_loop(1, _NCHUNK - 1, body, 0)

        g_last = _NCHUNK - 1
        s_last = g_last % 2
        gcopy(g_last, s_last).wait()
        wcopy(g_last, s_last).start()
        wcopy(g_last - 1, 1 - s_last).wait()
        wcopy(g_last, s_last).wait()

    return gather_kernel


def kernel(notation_ids, emb_table, W, b):
    P = _project_table(emb_table, W, b)
    ids_t = notation_ids.astype(jnp.int32).T.reshape(-1)
    out_planes = _make_gather()(P, ids_t)
    return out_planes.transpose(1, 0, 2)


# speedup vs baseline: 8.2218x; 1.0067x over previous
import functools

import jax
import jax.numpy as jnp
from jax import lax
from jax.experimental import pallas as pl
from jax.experimental.pallas import tpu as pltpu
from jax.experimental.pallas import tpu_sc as plsc

VOCAB = 1000
D = 512
_NROW = 4096
_L = 20
B_TOTAL = _NROW * _L


def _project_body(emb_ref, w_ref, b_ref, out_ref):
    p = lax.dot_general(
        emb_ref[...], w_ref[...],
        dimension_numbers=(((1,), (1,)), ((), ())),
        preferred_element_type=jnp.float32,
    )
    out_ref[...] = p + b_ref[...]


def _project_table(emb_table, W, b):
    return pl.pallas_call(
        _project_body,
        out_shape=jax.ShapeDtypeStruct((VOCAB, D), jnp.float32),
    )(emb_table, W, b.reshape(1, D))


_NW = 32
_B_PER_W = B_TOTAL // _NW
_CHUNK = 64
_NCHUNK = _B_PER_W // _CHUNK
_CPP = _NROW // _CHUNK
_NBUF = 3


def _make_gather():
    mesh = plsc.VectorSubcoreMesh(core_axis_name="c", subcore_axis_name="s")

    @functools.partial(
        pl.kernel,
        mesh=mesh,
        out_type=jax.ShapeDtypeStruct((_L, _NROW, D), jnp.float32),
        scratch_types=[
            pltpu.VMEM((_B_PER_W,), jnp.int32),
            pltpu.VMEM((_NBUF, _CHUNK, D), jnp.float32),
            pltpu.SemaphoreType.DMA,
            pltpu.SemaphoreType.DMA,
        ],
    )
    def gather_kernel(table_hbm, idx_hbm, out_hbm, idx_v, rows_v, gsem, wsem):
        wid = lax.axis_index("s") * 2 + lax.axis_index("c")
        base = wid * _B_PER_W
        kappa0 = wid * _NCHUNK
        pltpu.sync_copy(idx_hbm.at[pl.ds(base, _B_PER_W)], idx_v)

        def gcopy(g, slot):
            return pltpu.make_async_copy(
                table_hbm.at[idx_v.at[pl.ds(g * _CHUNK, _CHUNK)]],
                rows_v.at[slot], gsem)

        def wcopy(g, slot):
            kappa = kappa0 + g
            j = kappa // _CPP
            i0 = (kappa % _CPP) * _CHUNK
            return pltpu.make_async_copy(
                rows_v.at[slot],
                out_hbm.at[j, pl.ds(i0, _CHUNK)], wsem)

        for k in range(_NBUF):
            gcopy(k, k).start()

        def body(g, _):
            slot = g % _NBUF
            gcopy(g, slot).wait()
            wcopy(g, slot).start()
            wcopy(g, slot).wait()
            gcopy(g + _NBUF, slot).start()
            return 0

        lax.fori_loop(0, _NCHUNK - _NBUF, body, 0)

        def tail(g, _):
            slot = g % _NBUF
            gcopy(g, slot).wait()
            wcopy(g, slot).start()
            wcopy(g, slot).wait()
            return 0

        lax.fori_loop(_NCHUNK - _NBUF, _NCHUNK, tail, 0)

    return gather_kernel


def kernel(notation_ids, emb_table, W, b):
    P = _project_table(emb_table, W, b)
    ids_t = notation_ids.astype(jnp.int32).T.reshape(-1)
    out_planes = _make_gather()(P, ids_t)
    return out_planes.transpose(1, 0, 2)


# speedup vs baseline: 8.2408x; 1.0023x over previous
import functools

import jax
import jax.numpy as jnp
from jax import lax
from jax.experimental import pallas as pl
from jax.experimental.pallas import tpu as pltpu
from jax.experimental.pallas import tpu_sc as plsc

VOCAB = 1000
D = 512
_NROW = 4096
_L = 20
B_TOTAL = _NROW * _L


def _project_body(emb_ref, w_ref, b_ref, out_ref):
    p = lax.dot_general(
        emb_ref[...], w_ref[...],
        dimension_numbers=(((1,), (1,)), ((), ())),
        preferred_element_type=jnp.float32,
    )
    out_ref[...] = p + b_ref[...]


def _project_table(emb_table, W, b):
    return pl.pallas_call(
        _project_body,
        out_shape=jax.ShapeDtypeStruct((VOCAB, D), jnp.float32),
    )(emb_table, W, b.reshape(1, D))


_NW = 32
_B_PER_W = B_TOTAL // _NW
_CHUNK = 64
_NCHUNK = _B_PER_W // _CHUNK
_CPP = _NROW // _CHUNK
_NBUF = 3


def _make_gather():
    mesh = plsc.VectorSubcoreMesh(core_axis_name="c", subcore_axis_name="s")

    @functools.partial(
        pl.kernel,
        mesh=mesh,
        out_type=jax.ShapeDtypeStruct((_L, _NROW, D), jnp.float32),
        scratch_types=[
            pltpu.VMEM((_B_PER_W,), jnp.int32),
            pltpu.VMEM((_NBUF, _CHUNK, D), jnp.float32),
            pltpu.SemaphoreType.DMA,
            pltpu.SemaphoreType.DMA,
        ],
    )
    def gather_kernel(table_hbm, idx_hbm, out_hbm, idx_v, rows_v, gsem, wsem):
        wid = lax.axis_index("s") * 2 + lax.axis_index("c")
        base = wid * _B_PER_W
        kappa0 = wid * _NCHUNK
        pltpu.sync_copy(idx_hbm.at[pl.ds(base, _B_PER_W)], idx_v)

        def gcopy(g, slot):
            return pltpu.make_async_copy(
                table_hbm.at[idx_v.at[pl.ds(g * _CHUNK, _CHUNK)]],
                rows_v.at[slot], gsem)

        def wcopy(g, slot):
            kappa = kappa0 + g
            j = kappa // _CPP
            i0 = (kappa % _CPP) * _CHUNK
            return pltpu.make_async_copy(
                rows_v.at[slot],
                out_hbm.at[j, pl.ds(i0, _CHUNK)], wsem)

        gcopy(0, 0).start()
        gcopy(1, 1).start()

        gcopy(0, 0).wait()
        wcopy(0, 0).start()
        gcopy(2, 2).start()

        def body(g, _):
            slot = g % _NBUF
            gcopy(g, slot).wait()
            wcopy(g, slot).start()
            wcopy(g - 1, (g - 1) % _NBUF).wait()
            gcopy(g + 2, (g + 2) % _NBUF).start()
            return 0

        lax.fori_loop(1, _NCHUNK - 2, body, 0)

        for g in (_NCHUNK - 2, _NCHUNK - 1):
            gcopy(g, g % _NBUF).wait()
            wcopy(g, g % _NBUF).start()
            wcopy(g - 1, (g - 1) % _NBUF).wait()
        wcopy(_NCHUNK - 1, (_NCHUNK - 1) % _NBUF).wait()

    return gather_kernel


def kernel(notation_ids, emb_table, W, b):
    P = _project_table(emb_table, W, b)
    ids_t = notation_ids.astype(jnp.int32).T.reshape(-1)
    out_planes = _make_gather()(P, ids_t)
    return out_planes.transpose(1, 0, 2)
